# baseline (speedup 1.0000x reference)
import functools

import jax
import jax.numpy as jnp
from jax.experimental import pallas as pl
from jax.experimental.pallas import tpu as pltpu

_E = 64
_K = 6
_ALPHA = 0.001
_TILE = 8192


def _gate_body(w_ref, x_ref, idx_ref, w_out_ref, aux_ref, acc_ref, *, n_tokens):
    i = pl.program_id(0)
    n = pl.num_programs(0)

    @pl.when(i == 0)
    def _init():
        acc_ref[...] = jnp.zeros_like(acc_ref)

    w = w_ref[...]
    x = x_ref[...]
    logits = jax.lax.dot_general(
        w, x, (((1,), (1,)), ((), ())),
        preferred_element_type=jnp.float32)

    m = jnp.max(logits, axis=0, keepdims=True)
    ex = jnp.exp(logits - m)
    scores = ex / jnp.sum(ex, axis=0, keepdims=True)

    acc_ref[:, 0:1] += jnp.sum(scores, axis=1, keepdims=True)

    row = jax.lax.broadcasted_iota(jnp.int32, scores.shape, 0).astype(jnp.float32)
    work = scores
    idx_rows = []
    w_rows = []
    for _ in range(_K):
        mk = jnp.max(work, axis=0, keepdims=True)
        ik = jnp.min(jnp.where(work == mk, row, float(_E)),
                     axis=0, keepdims=True)
        work = jnp.where(row == ik, -1.0, work)
        idx_rows.append(ik)
        w_rows.append(mk)
    sel = (work < 0.0).astype(jnp.float32)
    acc_ref[:, 1:2] += jnp.sum(sel, axis=1, keepdims=True)

    idx_mat = jnp.concatenate(idx_rows + idx_rows[:2], axis=0)
    idx_t = jnp.transpose(idx_mat)
    idx_ref[...] = idx_t[:, :_K].astype(jnp.int32)
    w_mat = jnp.concatenate(w_rows + w_rows[:2], axis=0)
    denom = w_rows[0]
    for r in w_rows[1:]:
        denom = denom + r
    w_t = jnp.transpose(w_mat / (denom + 1e-20))
    w_out_ref[...] = w_t[:, :_K]

    @pl.when(i == n - 1)
    def _finish():
        pi = acc_ref[:, 0:1] * (1.0 / n_tokens)
        fi = acc_ref[:, 1:2] * (_E / (n_tokens * _K))
        aux_ref[...] = jnp.sum(pi * fi, axis=0, keepdims=True) * _ALPHA


def kernel(hidden_states, W):
    b, s, h = hidden_states.shape
    t = b * s
    x = hidden_states.reshape(t, h)
    tile = _TILE if t % _TILE == 0 else t
    grid = (t // tile,)

    body = functools.partial(_gate_body, n_tokens=t)
    idx_t, w_t, aux = pl.pallas_call(
        body,
        grid=grid,
        in_specs=[
            pl.BlockSpec((_E, h), lambda i: (0, 0)),
            pl.BlockSpec((tile, h), lambda i: (i, 0)),
        ],
        out_specs=[
            pl.BlockSpec((tile, _K), lambda i: (i, 0)),
            pl.BlockSpec((tile, _K), lambda i: (i, 0)),
            pl.BlockSpec((1, 1), lambda i: (0, 0)),
        ],
        out_shape=[
            jax.ShapeDtypeStruct((t, _K), jnp.int32),
            jax.ShapeDtypeStruct((t, _K), jnp.float32),
            jax.ShapeDtypeStruct((1, 1), jnp.float32),
        ],
        scratch_shapes=[pltpu.VMEM((_E, 2), jnp.float32)],
    )(W, x)
    return idx_t, w_t, aux.reshape(())


# speedup vs baseline: 2.0940x; 2.0940x over previous
import functools

import jax
import jax.numpy as jnp
from jax.experimental import pallas as pl
from jax.experimental.pallas import tpu as pltpu

_E = 64
_K = 6
_ALPHA = 0.001
_TILE = 16384


def _gate_body(w_ref, x_ref, idx_ref, w_out_ref, aux_ref, acc_ref, *, n_tokens):
    i = pl.program_id(0)
    n = pl.num_programs(0)

    @pl.when(i == 0)
    def _init():
        acc_ref[...] = jnp.zeros_like(acc_ref)

    w = w_ref[...]
    x = x_ref[...]
    logits = jax.lax.dot_general(
        w, x, (((1,), (1,)), ((), ())),
        preferred_element_type=jnp.float32)

    m = jnp.max(logits, axis=0, keepdims=True)
    ex = jnp.exp(logits - m)
    scores = ex / jnp.sum(ex, axis=0, keepdims=True)

    acc_ref[:, 0:1] += jnp.sum(scores, axis=1, keepdims=True)

    row = jax.lax.broadcasted_iota(jnp.int32, scores.shape, 0).astype(jnp.float32)
    work = scores
    idx_rows = []
    w_rows = []
    for _ in range(_K):
        mk = jnp.max(work, axis=0, keepdims=True)
        ik = jnp.min(jnp.where(work == mk, row, float(_E)),
                     axis=0, keepdims=True)
        work = jnp.where(row == ik, -1.0, work)
        idx_rows.append(ik)
        w_rows.append(mk)
    sel = (work < 0.0).astype(jnp.float32)
    acc_ref[:, 1:2] += jnp.sum(sel, axis=1, keepdims=True)

    idx_mat = jnp.concatenate(idx_rows, axis=0)
    idx_ref[...] = idx_mat.astype(jnp.int32)
    w_mat = jnp.concatenate(w_rows, axis=0)
    denom = w_rows[0]
    for r in w_rows[1:]:
        denom = denom + r
    w_out_ref[...] = w_mat / (denom + 1e-20)

    @pl.when(i == n - 1)
    def _finish():
        pi = acc_ref[:, 0:1] * (1.0 / n_tokens)
        fi = acc_ref[:, 1:2] * (_E / (n_tokens * _K))
        aux_ref[...] = jnp.sum(pi * fi, axis=0, keepdims=True) * _ALPHA


def kernel(hidden_states, W):
    b, s, h = hidden_states.shape
    t = b * s
    x = hidden_states.reshape(t, h)
    tile = _TILE if t % _TILE == 0 else t
    grid = (t // tile,)

    body = functools.partial(_gate_body, n_tokens=t)
    idx_t, w_t, aux = pl.pallas_call(
        body,
        grid=grid,
        in_specs=[
            pl.BlockSpec((_E, h), lambda i: (0, 0)),
            pl.BlockSpec((tile, h), lambda i: (i, 0)),
        ],
        out_specs=[
            pl.BlockSpec((_K, tile), lambda i: (0, i)),
            pl.BlockSpec((_K, tile), lambda i: (0, i)),
            pl.BlockSpec((1, 1), lambda i: (0, 0)),
        ],
        out_shape=[
            jax.ShapeDtypeStruct((_K, t), jnp.int32),
            jax.ShapeDtypeStruct((_K, t), jnp.float32),
            jax.ShapeDtypeStruct((1, 1), jnp.float32),
        ],
        scratch_shapes=[pltpu.VMEM((_E, 2), jnp.float32)],
    )(W, x)
    return idx_t.T, w_t.T, aux.reshape(())


# speedup vs baseline: 2.2362x; 1.0679x over previous
import functools

import jax
import jax.numpy as jnp
from jax.experimental import pallas as pl
from jax.experimental.pallas import tpu as pltpu

_E = 64
_K = 6
_ALPHA = 0.001
_TILE = 8192


def _gate_body(w_ref, x_ref, idx_ref, w_out_ref, aux_ref, acc_ref, *, n_tokens):
    i = pl.program_id(0)
    n = pl.num_programs(0)

    @pl.when(i == 0)
    def _init():
        acc_ref[...] = jnp.zeros_like(acc_ref)

    w = w_ref[...]
    x = x_ref[...]
    logits = jax.lax.dot_general(
        w, x, (((1,), (1,)), ((), ())),
        preferred_element_type=jnp.float32)

    m = jnp.max(logits, axis=0, keepdims=True)
    ex = jnp.exp(logits - m)
    zs = jnp.sum(ex, axis=0, keepdims=True)
    scores = ex / zs

    acc_ref[:, 0:1] += jnp.sum(scores, axis=1, keepdims=True)

    row = jax.lax.broadcasted_iota(jnp.int32, scores.shape, 0).astype(jnp.float32)
    work = scores
    idx_rows = []
    w_rows = []
    for k in range(_K):
        if k == 0:
            mk = 1.0 / zs
        else:
            mk = jnp.max(work, axis=0, keepdims=True)
        ik = jnp.min(jnp.where(work == mk, row, float(_E)),
                     axis=0, keepdims=True)
        work = jnp.where(row == ik, -1.0, work)
        idx_rows.append(ik)
        w_rows.append(mk)
    sel = (work < 0.0).astype(jnp.float32)
    acc_ref[:, 1:2] += jnp.sum(sel, axis=1, keepdims=True)

    idx_mat = jnp.concatenate(idx_rows, axis=0)
    idx_ref[...] = idx_mat.astype(jnp.int32)
    w_mat = jnp.concatenate(w_rows, axis=0)
    denom = w_rows[0]
    for r in w_rows[1:]:
        denom = denom + r
    w_out_ref[...] = w_mat / (denom + 1e-20)

    @pl.when(i == n - 1)
    def _finish():
        pi = acc_ref[:, 0:1] * (1.0 / n_tokens)
        fi = acc_ref[:, 1:2] * (_E / (n_tokens * _K))
        aux_ref[...] = jnp.sum(pi * fi, axis=0, keepdims=True) * _ALPHA


def kernel(hidden_states, W):
    b, s, h = hidden_states.shape
    t = b * s
    x = hidden_states.reshape(t, h)
    tile = _TILE if t % _TILE == 0 else t
    grid = (t // tile,)

    body = functools.partial(_gate_body, n_tokens=t)
    idx_t, w_t, aux = pl.pallas_call(
        body,
        grid=grid,
        in_specs=[
            pl.BlockSpec((_E, h), lambda i: (0, 0)),
            pl.BlockSpec((tile, h), lambda i: (i, 0)),
        ],
        out_specs=[
            pl.BlockSpec((_K, tile), lambda i: (0, i)),
            pl.BlockSpec((_K, tile), lambda i: (0, i)),
            pl.BlockSpec((1, 1), lambda i: (0, 0)),
        ],
        out_shape=[
            jax.ShapeDtypeStruct((_K, t), jnp.int32),
            jax.ShapeDtypeStruct((_K, t), jnp.float32),
            jax.ShapeDtypeStruct((1, 1), jnp.float32),
        ],
        scratch_shapes=[pltpu.VMEM((_E, 2), jnp.float32)],
    )(W, x)
    return idx_t.T, w_t.T, aux.reshape(())
